# baseline (speedup 1.0000x reference)
import functools

import jax
import jax.numpy as jnp
from jax import lax
from jax.experimental import pallas as pl
from jax.experimental.pallas import tpu as pltpu
from jax.experimental.pallas import tpu_sc as plsc

_NC = 2
_NS = 16
_NW = _NC * _NS
_NBUF = 4


def _emb_call(B, S, DP, s_per_w, idx, weight_p):
    mesh = plsc.VectorSubcoreMesh(core_axis_name="c", subcore_axis_name="s")
    n_per_w = s_per_w * S

    @functools.partial(
        pl.kernel,
        out_type=jax.ShapeDtypeStruct((B, S, DP), jnp.float32),
        mesh=mesh,
        scratch_types=[
            pltpu.VMEM((n_per_w,), jnp.int32),
            [pltpu.VMEM((S, DP), jnp.float32) for _ in range(_NBUF)],
            [pltpu.SemaphoreType.DMA for _ in range(_NBUF)],
        ],
    )
    def emb(idx_hbm, table_hbm, out_hbm, idx_v, rows, gsem):
        wid = lax.axis_index("s") * _NC + lax.axis_index("c")
        seq_base = wid * s_per_w

        pltpu.sync_copy(idx_hbm.at[pl.ds(seq_base * S, n_per_w)], idx_v)
        for b in range(_NBUF):
            pltpu.async_copy(
                table_hbm.at[idx_v.at[pl.ds(b * S, S)]], rows[b], gsem[b])

        def outer(jo, carry):
            i0 = jo * _NBUF
            for b in range(_NBUF):
                i = i0 + b
                pltpu.make_async_copy(
                    table_hbm.at[pl.ds(0, S)], rows[b], gsem[b]).wait()
                pltpu.sync_copy(rows[b], out_hbm.at[seq_base + i])

                @pl.when(i + _NBUF < s_per_w)
                def _():
                    nxt = i + _NBUF
                    pltpu.async_copy(
                        table_hbm.at[idx_v.at[pl.ds(nxt * S, S)]],
                        rows[b], gsem[b])
            return carry

        lax.fori_loop(0, s_per_w // _NBUF, outer, 0)

    return emb(idx, weight_p)


def kernel(input_ids, weight):
    B, S = input_ids.shape
    V, D = weight.shape
    DP = 128
    weight_p = jnp.pad(weight, ((0, 0), (0, DP - D)))
    idx = input_ids.reshape(B * S).astype(jnp.int32)

    s_per_w = B // _NW

    out = _emb_call(B, S, DP, s_per_w, idx, weight_p)
    return out[:, :, :D]


# speedup vs baseline: 1.0014x; 1.0014x over previous
import functools

import jax
import jax.numpy as jnp
from jax import lax
from jax.experimental import pallas as pl
from jax.experimental.pallas import tpu as pltpu
from jax.experimental.pallas import tpu_sc as plsc

_NC = 2
_NS = 16
_NW = _NC * _NS
_NBUF = 2
_CH = 400


def _emb_call(N, DP, n_per_w, idx, weight_p):
    mesh = plsc.VectorSubcoreMesh(core_axis_name="c", subcore_axis_name="s")
    n_ch = n_per_w // _CH

    @functools.partial(
        pl.kernel,
        out_type=jax.ShapeDtypeStruct((N, DP), jnp.float32),
        mesh=mesh,
        scratch_types=[
            pltpu.VMEM((n_per_w,), jnp.int32),
            [pltpu.VMEM((_CH, DP), jnp.float32) for _ in range(_NBUF)],
            [pltpu.SemaphoreType.DMA for _ in range(_NBUF)],
        ],
    )
    def emb(idx_hbm, table_hbm, out_hbm, idx_v, rows, gsem):
        wid = lax.axis_index("s") * _NC + lax.axis_index("c")
        base = wid * n_per_w

        pltpu.sync_copy(idx_hbm.at[pl.ds(base, n_per_w)], idx_v)
        for b in range(_NBUF):
            pltpu.async_copy(
                table_hbm.at[idx_v.at[pl.ds(b * _CH, _CH)]], rows[b], gsem[b])

        def outer(jo, carry):
            i0 = jo * _NBUF
            for b in range(_NBUF):
                i = i0 + b
                pltpu.make_async_copy(
                    table_hbm.at[pl.ds(0, _CH)], rows[b], gsem[b]).wait()
                pltpu.sync_copy(rows[b], out_hbm.at[pl.ds(base + i * _CH, _CH)])

                @pl.when(i + _NBUF < n_ch)
                def _():
                    nxt = i + _NBUF
                    pltpu.async_copy(
                        table_hbm.at[idx_v.at[pl.ds(nxt * _CH, _CH)]],
                        rows[b], gsem[b])
            return carry

        lax.fori_loop(0, n_ch // _NBUF, outer, 0)

    return emb(idx, weight_p)


def kernel(input_ids, weight):
    B, S = input_ids.shape
    V, D = weight.shape
    DP = 128
    weight_p = jnp.pad(weight, ((0, 0), (0, DP - D)))
    N = B * S
    idx = input_ids.reshape(N).astype(jnp.int32)

    n_per_w = N // _NW

    out = _emb_call(N, DP, n_per_w, idx, weight_p)
    return out[:, :D].reshape(B, S, D)


# speedup vs baseline: 1.3501x; 1.3481x over previous
import functools

import jax
import jax.numpy as jnp
from jax import lax
from jax.experimental import pallas as pl
from jax.experimental.pallas import tpu as pltpu
from jax.experimental.pallas import tpu_sc as plsc

_NC = 2
_NS = 16
_NW = _NC * _NS
_NBUF = 4


def _emb_call(B, S, D, DP, s_per_w, idx, weight):
    mesh = plsc.VectorSubcoreMesh(core_axis_name="c", subcore_axis_name="s")
    n_per_w = s_per_w * S

    @functools.partial(
        pl.kernel,
        out_type=jax.ShapeDtypeStruct((B, S, DP), jnp.float32),
        mesh=mesh,
        scratch_types=[
            pltpu.VMEM((n_per_w,), jnp.int32),
            [pltpu.VMEM((S, D), jnp.float32) for _ in range(_NBUF)],
            [pltpu.SemaphoreType.DMA for _ in range(_NBUF)],
        ],
        compiler_params=pltpu.CompilerParams(use_tc_tiling_on_sc=False),
    )
    def emb(idx_hbm, table_hbm, out_hbm, idx_v, rows, gsem):
        wid = lax.axis_index("s") * _NC + lax.axis_index("c")
        seq_base = wid * s_per_w

        pltpu.sync_copy(idx_hbm.at[pl.ds(seq_base * S, n_per_w)], idx_v)
        for b in range(_NBUF):
            pltpu.async_copy(
                table_hbm.at[idx_v.at[pl.ds(b * S, S)]], rows[b], gsem[b])

        def outer(jo, carry):
            i0 = jo * _NBUF
            for b in range(_NBUF):
                i = i0 + b
                pltpu.make_async_copy(
                    table_hbm.at[pl.ds(0, S)], rows[b], gsem[b]).wait()
                pltpu.sync_copy(
                    rows[b], out_hbm.at[seq_base + i, :, pl.ds(0, D)])

                @pl.when(i + _NBUF < s_per_w)
                def _():
                    nxt = i + _NBUF
                    pltpu.async_copy(
                        table_hbm.at[idx_v.at[pl.ds(nxt * S, S)]],
                        rows[b], gsem[b])
            return carry

        lax.fori_loop(0, s_per_w // _NBUF, outer, 0)

    return emb(idx, weight)


def kernel(input_ids, weight):
    B, S = input_ids.shape
    V, D = weight.shape
    DP = 128
    idx = input_ids.reshape(B * S).astype(jnp.int32)

    s_per_w = B // _NW

    out = _emb_call(B, S, D, DP, s_per_w, idx, weight)
    return out[:, :, :D]
